# baseline (speedup 1.0000x reference)
import functools
import jax
import jax.numpy as jnp
from jax import lax
from jax.experimental import pallas as pl
from jax.experimental.pallas import tpu as pltpu, tpu_sc as plsc

N = 10000
E = 320000
NC = 2
NS = 16
CB = 120
IB = 8
RPT = 632
RPT_LAST = N - (NS - 1) * RPT
N_PAD = N + 8


def _nch(ept):
  return -(-(-(-ept // CB)) // IB) * IB


def _copy_rows(src_ref, dst_ref, s):
  @pl.when(s < NS - 1)
  def _():
    pltpu.sync_copy(src_ref.at[pl.ds(s * RPT, RPT)],
                    dst_ref.at[pl.ds(s * RPT, RPT)])

  @pl.when(s == NS - 1)
  def _():
    pltpu.sync_copy(src_ref.at[pl.ds((NS - 1) * RPT, RPT_LAST)],
                    dst_ref.at[pl.ds((NS - 1) * RPT, RPT_LAST)])


def _sc_agg_body(h0, h1, src_r, dst_r, out0, out1,
                 srcv, dstv, rows0, rows1, rows2, acc,
                 gsem0, gsem1, gsem2, ssem0, ssem1, ssem2, nch, edge_split):
  c = lax.axis_index("c")
  s = lax.axis_index("s")
  t = c * NS + s if edge_split else s

  @pl.when(c == 0)
  def _():
    _copy_rows(h0, acc, s)

  @pl.when(c == 1)
  def _():
    _copy_rows(h1, acc, s)

  plsc.subcore_barrier()

  def start_gather(k, buf, sem):
    @pl.when(c == 0)
    def _():
      pltpu.async_copy(h0.at[srcv.at[k]], buf, sem)

    @pl.when(c == 1)
    def _():
      pltpu.async_copy(h1.at[srcv.at[k]], buf, sem)

  def wait_gather(buf, sem):
    pltpu.make_async_copy(h0.at[pl.ds(0, CB)], buf, sem).wait()

  bufs = ((rows0, gsem0), (rows1, gsem1), (rows2, gsem2))
  ssems = (ssem0, ssem1, ssem2)

  def wait_scatter(b):
    pltpu.make_async_copy(bufs[b][0], acc.at[pl.ds(0, CB)], ssems[b]).wait()

  def stage(st, carry):
    @pl.when(st > 0)
    def _():
      wait_scatter(0)
      wait_scatter(1)
      wait_scatter(2)

    pltpu.sync_copy(src_r.at[t, pl.ds(st * IB, IB)], srcv)
    pltpu.sync_copy(dst_r.at[t, pl.ds(st * IB, IB)], dstv)

    start_gather(0, *bufs[0])
    start_gather(1, *bufs[1])
    for k in range(IB):
      buf, sem = bufs[k % 3]
      if k + 2 < IB:
        if k >= 1:
          wait_scatter((k + 2) % 3)
        start_gather(k + 2, *bufs[(k + 2) % 3])
      wait_gather(buf, sem)
      pltpu.async_copy(buf, acc.at[dstv.at[k]], ssems[k % 3], add=True)
    return carry

  lax.fori_loop(0, nch // IB, stage, 0)

  wait_scatter(0)
  wait_scatter(1)
  wait_scatter(2)

  plsc.subcore_barrier()

  @pl.when(c == 0)
  def _():
    _copy_rows(acc, out0, s)

  @pl.when(c == 1)
  def _():
    _copy_rows(acc, out1, s)


@functools.partial(jax.jit, static_argnames=("nch", "edge_split"))
def _sc_agg(h0, h1, src_r, dst_r, nch, edge_split):
  mesh = plsc.VectorSubcoreMesh(core_axis_name="c", subcore_axis_name="s",
                                num_cores=NC, num_subcores=NS)
  F = h0.shape[1]
  return pl.kernel(
      functools.partial(_sc_agg_body, nch=nch, edge_split=edge_split),
      out_type=(jax.ShapeDtypeStruct((N, F), jnp.float32),
                jax.ShapeDtypeStruct((N, F), jnp.float32)),
      mesh=mesh,
      scratch_types=[
          pltpu.VMEM((IB, CB), jnp.int32),
          pltpu.VMEM((IB, CB), jnp.int32),
          pltpu.VMEM((CB, F), jnp.float32),
          pltpu.VMEM((CB, F), jnp.float32),
          pltpu.VMEM((CB, F), jnp.float32),
          pltpu.VMEM_SHARED((N_PAD, F), jnp.float32),
          pltpu.SemaphoreType.DMA,
          pltpu.SemaphoreType.DMA,
          pltpu.SemaphoreType.DMA,
          pltpu.SemaphoreType.DMA,
          pltpu.SemaphoreType.DMA,
          pltpu.SemaphoreType.DMA,
      ],
  )(h0, h1, src_r, dst_r)


def _pad_edges(idx, parts, fill):
  ept = E // parts
  nch = _nch(ept)
  pad = nch * CB - ept
  return jnp.pad(idx.reshape(parts, ept), ((0, 0), (0, pad)),
                 constant_values=fill).reshape(parts, nch, CB), nch


def _mlp_body0(h0_ref, h1_ref, xm_ref, w1_ref, b1_ref, w2_ref, b2_ref,
               o0_ref, o1_ref):
  g = h0_ref[...] + h1_ref[...] - xm_ref[...]
  h = jnp.dot(g, w1_ref[...], preferred_element_type=jnp.float32)
  h = jnp.maximum(h + b1_ref[...], 0.0)
  h = jnp.dot(h, w2_ref[...], preferred_element_type=jnp.float32)
  h = jnp.maximum(h + b2_ref[...], 0.0)
  half = h.shape[1] // 2
  o0_ref[...] = h[:, :half]
  o1_ref[...] = h[:, half:]


def _mlp_body(h0_ref, h1_ref, w1a_ref, w1b_ref, b1_ref, w2_ref, b2_ref,
              o0_ref, o1_ref):
  h = jnp.dot(h0_ref[...], w1a_ref[...], preferred_element_type=jnp.float32)
  h = h + jnp.dot(h1_ref[...], w1b_ref[...],
                  preferred_element_type=jnp.float32)
  h = jnp.maximum(h + b1_ref[...], 0.0)
  h = jnp.dot(h, w2_ref[...], preferred_element_type=jnp.float32)
  h = jnp.maximum(h + b2_ref[...], 0.0)
  half = h.shape[1] // 2
  o0_ref[...] = h[:, :half]
  o1_ref[...] = h[:, half:]


@functools.partial(jax.jit, static_argnames=("sub_x",))
def _mlp(h0, h1, xm, w1a, w1b, b1, w2, b2, sub_x):
  BN = 1000
  F = h0.shape[1]
  H = w2.shape[0]
  grid = (N // BN,)
  row_spec = pl.BlockSpec((BN, F), lambda i: (i, 0))
  if sub_x:
    body = _mlp_body0
    operands = (h0, h1, xm, w1a, b1, w2, b2)
    in_specs = [row_spec, row_spec, row_spec,
                pl.BlockSpec((F, H), lambda i: (0, 0)),
                pl.BlockSpec((1, H), lambda i: (0, 0)),
                pl.BlockSpec((H, H), lambda i: (0, 0)),
                pl.BlockSpec((1, H), lambda i: (0, 0))]
  else:
    body = _mlp_body
    operands = (h0, h1, w1a, w1b, b1, w2, b2)
    in_specs = [row_spec, row_spec,
                pl.BlockSpec((F, H), lambda i: (0, 0)),
                pl.BlockSpec((F, H), lambda i: (0, 0)),
                pl.BlockSpec((1, H), lambda i: (0, 0)),
                pl.BlockSpec((H, H), lambda i: (0, 0)),
                pl.BlockSpec((1, H), lambda i: (0, 0))]
  return pl.pallas_call(
      body,
      grid=grid,
      in_specs=in_specs,
      out_specs=[
          pl.BlockSpec((BN, H // 2), lambda i: (i, 0)),
          pl.BlockSpec((BN, H // 2), lambda i: (i, 0)),
      ],
      out_shape=[
          jax.ShapeDtypeStruct((N, H // 2), jnp.float32),
          jax.ShapeDtypeStruct((N, H // 2), jnp.float32),
      ],
  )(*operands)


def kernel(x, edge_index, W1_0, b1_0, W2_0, b2_0, W1_1, b1_1, W2_1, b2_1,
           W1_2, b1_2, W2_2, b2_2):
  scale = 1.0 / jnp.sqrt(jnp.float32(1.0 + 1e-5))

  key = jnp.sort(edge_index[0].astype(jnp.int32) * 16384
                 + edge_index[1].astype(jnp.int32))
  src = key >> 14
  dst = key & 16383
  src_e, nch_e = _pad_edges(src, NC * NS, 0)
  dst_e, _ = _pad_edges(dst, NC * NS, N)
  src_f, nch_f = _pad_edges(src, NS, 0)
  dst_f, _ = _pad_edges(dst, NS, N)

  params = [(W1_0, b1_0, W2_0, b2_0), (W1_1, b1_1, W2_1, b2_1),
            (W1_2, b1_2, W2_2, b2_2)]

  h0 = h1 = x
  for i in range(3):
    W1, b1, W2, b2 = params[i]
    w1s = W1 * scale
    b1s = (b1 * scale).reshape(1, -1)
    w2s = W2 * scale
    b2s = (b2 * scale).reshape(1, -1)
    F = W1.shape[0] if i == 0 else W1.shape[0] // 2
    if i == 0:
      a0, a1 = _sc_agg(h0, h1, src_e, dst_e, nch=nch_e, edge_split=True)
      h0, h1 = _mlp(a0, a1, x, w1s, w1s, b1s, w2s, b2s, sub_x=True)
    else:
      a0, a1 = _sc_agg(h0, h1, src_f, dst_f, nch=nch_f, edge_split=False)
      h0, h1 = _mlp(a0, a1, None, w1s[:F], w1s[F:], b1s, w2s, b2s,
                    sub_x=False)

  return jnp.concatenate([h0, h1], axis=1)


# speedup vs baseline: 1.5512x; 1.5512x over previous
import functools
import jax
import jax.numpy as jnp
from jax import lax
from jax.experimental import pallas as pl
from jax.experimental.pallas import tpu as pltpu, tpu_sc as plsc

N = 10000
E = 320000
NC = 2
NS = 16
CB = 120
IB = 8
RPT = 632
RPT_LAST = N - (NS - 1) * RPT
N_PAD = N + 8


def _nch(ept):
  return -(-(-(-ept // CB)) // IB) * IB


def _copy_rows(src_ref, dst_ref, s):
  @pl.when(s < NS - 1)
  def _():
    pltpu.sync_copy(src_ref.at[pl.ds(s * RPT, RPT)],
                    dst_ref.at[pl.ds(s * RPT, RPT)])

  @pl.when(s == NS - 1)
  def _():
    pltpu.sync_copy(src_ref.at[pl.ds((NS - 1) * RPT, RPT_LAST)],
                    dst_ref.at[pl.ds((NS - 1) * RPT, RPT_LAST)])


def _sc_agg_body(h0, h1, comb_r, out0, out1,
                 idxv0, idxv1, rows0, rows1, rows2, acc,
                 gsem0, gsem1, gsem2, ssem0, ssem1, ssem2, xsem,
                 nch, edge_split):
  c = lax.axis_index("c")
  s = lax.axis_index("s")
  t = c * NS + s if edge_split else s
  nst = nch // IB

  @pl.when(c == 0)
  def _():
    _copy_rows(h0, acc, s)

  @pl.when(c == 1)
  def _():
    _copy_rows(h1, acc, s)

  plsc.subcore_barrier()

  def start_gather(k, iv, buf, sem):
    @pl.when(c == 0)
    def _():
      pltpu.async_copy(h0.at[iv.at[2 * k]], buf, sem)

    @pl.when(c == 1)
    def _():
      pltpu.async_copy(h1.at[iv.at[2 * k]], buf, sem)

  def wait_gather(buf, sem):
    pltpu.make_async_copy(h0.at[pl.ds(0, CB)], buf, sem).wait()

  bufs = ((rows0, gsem0), (rows1, gsem1), (rows2, gsem2))
  ssems = (ssem0, ssem1, ssem2)

  def wait_scatter(b):
    pltpu.make_async_copy(bufs[b][0], acc.at[pl.ds(0, CB)], ssems[b]).wait()

  pltpu.sync_copy(comb_r.at[t, pl.ds(0, 2 * IB)], idxv0)

  def stage_impl(st, iv, niv):
    @pl.when(st > 0)
    def _():
      wait_scatter(0)
      wait_scatter(1)
      wait_scatter(2)
      pltpu.make_async_copy(comb_r.at[t, pl.ds(0, 2 * IB)], iv, xsem).wait()

    @pl.when(st + 1 < nst)
    def _():
      pltpu.async_copy(comb_r.at[t, pl.ds((st + 1) * 2 * IB, 2 * IB)],
                       niv, xsem)

    start_gather(0, iv, *bufs[0])
    start_gather(1, iv, *bufs[1])
    for k in range(IB):
      buf, sem = bufs[k % 3]
      if k + 2 < IB:
        if k >= 1:
          wait_scatter((k + 2) % 3)
        start_gather(k + 2, iv, *bufs[(k + 2) % 3])
      wait_gather(buf, sem)
      pltpu.async_copy(buf, acc.at[iv.at[2 * k + 1]], ssems[k % 3], add=True)

  def stage(st, carry):
    @pl.when(st % 2 == 0)
    def _():
      stage_impl(st, idxv0, idxv1)

    @pl.when(st % 2 == 1)
    def _():
      stage_impl(st, idxv1, idxv0)
    return carry

  lax.fori_loop(0, nst, stage, 0)

  wait_scatter(0)
  wait_scatter(1)
  wait_scatter(2)

  plsc.subcore_barrier()

  @pl.when(c == 0)
  def _():
    _copy_rows(acc, out0, s)

  @pl.when(c == 1)
  def _():
    _copy_rows(acc, out1, s)


@functools.partial(jax.jit, static_argnames=("nch", "edge_split"))
def _sc_agg(h0, h1, comb_r, nch, edge_split):
  mesh = plsc.VectorSubcoreMesh(core_axis_name="c", subcore_axis_name="s",
                                num_cores=NC, num_subcores=NS)
  F = h0.shape[1]
  return pl.kernel(
      functools.partial(_sc_agg_body, nch=nch, edge_split=edge_split),
      out_type=(jax.ShapeDtypeStruct((N, F), jnp.float32),
                jax.ShapeDtypeStruct((N, F), jnp.float32)),
      mesh=mesh,
      scratch_types=[
          pltpu.VMEM((2 * IB, CB), jnp.int32),
          pltpu.VMEM((2 * IB, CB), jnp.int32),
          pltpu.VMEM((CB, F), jnp.float32),
          pltpu.VMEM((CB, F), jnp.float32),
          pltpu.VMEM((CB, F), jnp.float32),
          pltpu.VMEM_SHARED((N_PAD, F), jnp.float32),
          pltpu.SemaphoreType.DMA,
          pltpu.SemaphoreType.DMA,
          pltpu.SemaphoreType.DMA,
          pltpu.SemaphoreType.DMA,
          pltpu.SemaphoreType.DMA,
          pltpu.SemaphoreType.DMA,
          pltpu.SemaphoreType.DMA,
      ],
  )(h0, h1, comb_r)


def _pad_edges(src, dst, parts):
  ept = E // parts
  nch = _nch(ept)
  pad = nch * CB - ept
  src_r = jnp.pad(src.reshape(parts, ept), ((0, 0), (0, pad))
                  ).reshape(parts, nch, CB)
  dst_r = jnp.pad(dst.reshape(parts, ept), ((0, 0), (0, pad)),
                  constant_values=N).reshape(parts, nch, CB)
  comb = jnp.stack([src_r, dst_r], axis=2).reshape(parts, 2 * nch, CB)
  return comb, nch


def _mlp_body0(h0_ref, h1_ref, xm_ref, w1_ref, b1_ref, w2_ref, b2_ref,
               o0_ref, o1_ref):
  g = h0_ref[...] + h1_ref[...] - xm_ref[...]
  h = jnp.dot(g, w1_ref[...], preferred_element_type=jnp.float32)
  h = jnp.maximum(h + b1_ref[...], 0.0)
  h = jnp.dot(h, w2_ref[...], preferred_element_type=jnp.float32)
  h = jnp.maximum(h + b2_ref[...], 0.0)
  half = h.shape[1] // 2
  o0_ref[...] = h[:, :half]
  o1_ref[...] = h[:, half:]


def _mlp_body(h0_ref, h1_ref, w1a_ref, w1b_ref, b1_ref, w2_ref, b2_ref,
              o0_ref, o1_ref):
  h = jnp.dot(h0_ref[...], w1a_ref[...], preferred_element_type=jnp.float32)
  h = h + jnp.dot(h1_ref[...], w1b_ref[...],
                  preferred_element_type=jnp.float32)
  h = jnp.maximum(h + b1_ref[...], 0.0)
  h = jnp.dot(h, w2_ref[...], preferred_element_type=jnp.float32)
  h = jnp.maximum(h + b2_ref[...], 0.0)
  half = h.shape[1] // 2
  o0_ref[...] = h[:, :half]
  o1_ref[...] = h[:, half:]


@functools.partial(jax.jit, static_argnames=("sub_x",))
def _mlp(h0, h1, xm, w1a, w1b, b1, w2, b2, sub_x):
  BN = 1000
  F = h0.shape[1]
  H = w2.shape[0]
  grid = (N // BN,)
  row_spec = pl.BlockSpec((BN, F), lambda i: (i, 0))
  if sub_x:
    body = _mlp_body0
    operands = (h0, h1, xm, w1a, b1, w2, b2)
    in_specs = [row_spec, row_spec, row_spec,
                pl.BlockSpec((F, H), lambda i: (0, 0)),
                pl.BlockSpec((1, H), lambda i: (0, 0)),
                pl.BlockSpec((H, H), lambda i: (0, 0)),
                pl.BlockSpec((1, H), lambda i: (0, 0))]
  else:
    body = _mlp_body
    operands = (h0, h1, w1a, w1b, b1, w2, b2)
    in_specs = [row_spec, row_spec,
                pl.BlockSpec((F, H), lambda i: (0, 0)),
                pl.BlockSpec((F, H), lambda i: (0, 0)),
                pl.BlockSpec((1, H), lambda i: (0, 0)),
                pl.BlockSpec((H, H), lambda i: (0, 0)),
                pl.BlockSpec((1, H), lambda i: (0, 0))]
  return pl.pallas_call(
      body,
      grid=grid,
      in_specs=in_specs,
      out_specs=[
          pl.BlockSpec((BN, H // 2), lambda i: (i, 0)),
          pl.BlockSpec((BN, H // 2), lambda i: (i, 0)),
      ],
      out_shape=[
          jax.ShapeDtypeStruct((N, H // 2), jnp.float32),
          jax.ShapeDtypeStruct((N, H // 2), jnp.float32),
      ],
  )(*operands)


def kernel(x, edge_index, W1_0, b1_0, W2_0, b2_0, W1_1, b1_1, W2_1, b2_1,
           W1_2, b1_2, W2_2, b2_2):
  scale = 1.0 / jnp.sqrt(jnp.float32(1.0 + 1e-5))

  src = edge_index[0].astype(jnp.int32)
  dst = edge_index[1].astype(jnp.int32)
  comb_e, nch_e = _pad_edges(src, dst, NC * NS)
  comb_f, nch_f = _pad_edges(src, dst, NS)

  params = [(W1_0, b1_0, W2_0, b2_0), (W1_1, b1_1, W2_1, b2_1),
            (W1_2, b1_2, W2_2, b2_2)]

  h0 = h1 = x
  for i in range(3):
    W1, b1, W2, b2 = params[i]
    w1s = W1 * scale
    b1s = (b1 * scale).reshape(1, -1)
    w2s = W2 * scale
    b2s = (b2 * scale).reshape(1, -1)
    F = W1.shape[0] if i == 0 else W1.shape[0] // 2
    if i == 0:
      a0, a1 = _sc_agg(h0, h1, comb_e, nch=nch_e, edge_split=True)
      h0, h1 = _mlp(a0, a1, x, w1s, w1s, b1s, w2s, b2s, sub_x=True)
    else:
      a0, a1 = _sc_agg(h0, h1, comb_f, nch=nch_f, edge_split=False)
      h0, h1 = _mlp(a0, a1, None, w1s[:F], w1s[F:], b1s, w2s, b2s,
                    sub_x=False)

  return jnp.concatenate([h0, h1], axis=1)
